# baseline (speedup 1.0000x reference)
import functools

import jax
import jax.numpy as jnp
from jax import lax
from jax.experimental import pallas as pl
from jax.experimental.pallas import tpu as pltpu
from jax.experimental.pallas import tpu_sc as plsc

VOCAB_SIZE = 1000
DIM = 64
DIM2 = 128
B = 16384
H = B // 2

_info = plsc.get_sparse_core_info()
_NC, _NS = _info.num_cores, _info.num_subcores
_NW = _NC * _NS
_QPW = H // _NW
_CHUNK = 128
_NCHUNK = _QPW // _CHUNK


def _body(idx_hbm, table_hbm, out_hbm, idx_v, rows_v, sem):
    wid = lax.axis_index("s") * _NC + lax.axis_index("c")
    qbase = wid * _QPW
    pltpu.sync_copy(idx_hbm.at[pl.ds(qbase, _QPW)], idx_v.at[pl.ds(0, _QPW)])
    pltpu.sync_copy(
        idx_hbm.at[pl.ds(H + qbase, _QPW)], idx_v.at[pl.ds(_QPW, _QPW)]
    )
    copies = [
        pltpu.async_copy(
            table_hbm.at[idx_v.at[pl.ds(h * _QPW + c * _CHUNK, _CHUNK)]],
            rows_v.at[h, pl.ds(c * _CHUNK, _CHUNK)],
            sem,
        )
        for h in range(2)
        for c in range(_NCHUNK)
    ]
    for c in copies:
        c.wait()
    for h in range(2):
        pltpu.sync_copy(
            rows_v.at[h],
            out_hbm.at[pl.ds(qbase, _QPW), pl.ds(h * DIM, DIM)],
        )


@functools.partial(jax.jit, static_argnames=())
def kernel(indices, table):
    idx = indices.astype(jnp.int32)
    run = pl.kernel(
        _body,
        out_type=jax.ShapeDtypeStruct((H, DIM2), jnp.float32),
        mesh=plsc.VectorSubcoreMesh(core_axis_name="c", subcore_axis_name="s"),
        scratch_types=[
            pltpu.VMEM((2 * _QPW,), jnp.int32),
            pltpu.VMEM((2, _QPW, DIM), jnp.float32),
            pltpu.SemaphoreType.DMA,
        ],
        compiler_params=pltpu.CompilerParams(use_tc_tiling_on_sc=False),
    )
    packed = run(idx, table)
    return jnp.concatenate([packed[:, :DIM], packed[:, DIM:]], axis=0)


# speedup vs baseline: 1.3582x; 1.3582x over previous
import functools

import jax
import jax.numpy as jnp
from jax import lax
from jax.experimental import pallas as pl
from jax.experimental.pallas import tpu as pltpu
from jax.experimental.pallas import tpu_sc as plsc

VOCAB_SIZE = 1000
DIM = 64
DIM_PAD = 128
B = 16384

_info = plsc.get_sparse_core_info()
_NC, _NS = _info.num_cores, _info.num_subcores
_NW = _NC * _NS
_BPW = B // _NW
_CHUNK = 128
_NCHUNK = _BPW // _CHUNK


def _body(idx_hbm, table_hbm, out_hbm, idx_v, rows_v, sem):
    wid = lax.axis_index("s") * _NC + lax.axis_index("c")
    base = wid * _BPW
    pltpu.sync_copy(idx_hbm.at[pl.ds(base, _BPW)], idx_v)
    copies = [
        pltpu.async_copy(
            table_hbm.at[idx_v.at[pl.ds(j * _CHUNK, _CHUNK)]],
            rows_v.at[pl.ds(j * _CHUNK, _CHUNK)],
            sem,
        )
        for j in range(_NCHUNK)
    ]
    for c in copies:
        c.wait()
    pltpu.sync_copy(
        rows_v,
        out_hbm.at[pl.ds(base, _BPW), pl.ds(0, DIM)],
    )


@functools.partial(jax.jit, static_argnames=())
def kernel(indices, table):
    idx = indices.astype(jnp.int32)
    run = pl.kernel(
        _body,
        out_type=jax.ShapeDtypeStruct((B, DIM_PAD), jnp.float32),
        mesh=plsc.VectorSubcoreMesh(core_axis_name="c", subcore_axis_name="s"),
        scratch_types=[
            pltpu.VMEM((_BPW,), jnp.int32),
            pltpu.VMEM((_BPW, DIM), jnp.float32),
            pltpu.SemaphoreType.DMA,
        ],
        compiler_params=pltpu.CompilerParams(use_tc_tiling_on_sc=False),
    )
    return run(idx, table)[:, :DIM]
